# baseline (speedup 1.0000x reference)
import functools

import jax
import jax.numpy as jnp
from jax import lax
from jax.experimental import pallas as pl
from jax.experimental.pallas import tpu as pltpu
from jax.experimental.pallas import tpu_sc as plsc

_L = 16
_T = 1001
_T_PAD = 1024
_N_CHUNKS = _T_PAD // _L
_B = 16384
_NW = 16
_B_PER_W = _B // _NW
_G_CHUNKS = _B_PER_W // _L
_BUILD_UNROLL = 4
_GATHER_UNROLL = 8

_mesh = plsc.VectorSubcoreMesh(core_axis_name="c", subcore_axis_name="s", num_cores=1)


@functools.partial(
    pl.kernel,
    mesh=_mesh,
    out_type=jax.ShapeDtypeStruct((_B,), jnp.float32),
    compiler_params=pltpu.CompilerParams(needs_layout_passes=False),
    scratch_types=[
        pltpu.VMEM((_T_PAD,), jnp.float32),
        pltpu.VMEM((2 * _L,), jnp.float32),
        pltpu.VMEM((_B_PER_W,), jnp.int32),
        pltpu.VMEM((_B_PER_W,), jnp.float32),
        pltpu.SemaphoreType.DMA,
        pltpu.SemaphoreType.DMA,
        pltpu.SemaphoreType.DMA,
    ],
)
def _snr_lookup(ts_hbm, w_hbm, wini_hbm, md_hbm, bini_hbm, base_hbm, out_hbm,
                tab_v, scal_v, idx_v, out_v, sem0, sem1, sem2):
    wid = lax.axis_index("s")
    off = wid * _B_PER_W
    cw = pltpu.async_copy(w_hbm, tab_v.at[pl.ds(0, _T)], sem0)
    c0 = pltpu.async_copy(wini_hbm, scal_v.at[pl.ds(0, 1)], sem1)
    c1 = pltpu.async_copy(md_hbm, scal_v.at[pl.ds(8, 1)], sem1)
    c2 = pltpu.async_copy(bini_hbm, scal_v.at[pl.ds(16, 1)], sem1)
    c3 = pltpu.async_copy(base_hbm, scal_v.at[pl.ds(24, 1)], sem1)
    ci = pltpu.async_copy(ts_hbm.at[pl.ds(off, _B_PER_W)], idx_v, sem2)

    c0.wait()
    c1.wait()
    c2.wait()
    c3.wait()
    sv0 = scal_v[pl.ds(0, _L)]
    sv1 = scal_v[pl.ds(_L, _L)]
    wini = lax.broadcast(sv0[0], (_L,))
    md = lax.broadcast(sv0[8], (_L,))
    carry0 = lax.broadcast(-(sv1[0] + sv1[8]), (_L,))
    cw.wait()

    @plsc.parallel_loop(0, _N_CHUNKS, 1, unroll=_BUILD_UNROLL, carry=carry0)
    def _build(i, carry):
        x = tab_v[pl.ds(i * _L, _L)] + wini
        s = 1.0 / (1.0 + jnp.exp(-x)) + md
        tab_v[pl.ds(i * _L, _L)] = plsc.cumsum(s) + carry
        return carry + lax.broadcast(jnp.sum(s), (_L,))

    ci.wait()

    @plsc.parallel_loop(0, _G_CHUNKS, 1, unroll=_GATHER_UNROLL)
    def _gather(j):
        ids = idx_v[pl.ds(j * _L, _L)]
        out_v[pl.ds(j * _L, _L)] = plsc.load_gather(tab_v, [ids])

    pltpu.sync_copy(out_v, out_hbm.at[pl.ds(off, _B_PER_W)])


def kernel(timesteps, W, base, w_ini, base_ini, min_diff):
    as1 = lambda x: jnp.asarray(x, jnp.float32).reshape(1)
    return _snr_lookup(timesteps, W, as1(w_ini), as1(min_diff),
                       as1(base_ini), as1(base[0]))


# speedup vs baseline: 1.0055x; 1.0055x over previous
import functools

import jax
import jax.numpy as jnp
from jax import lax
from jax.experimental import pallas as pl
from jax.experimental.pallas import tpu as pltpu
from jax.experimental.pallas import tpu_sc as plsc

_L = 16
_T = 1001
_T_PAD = 1024
_N_CHUNKS = _T_PAD // _L
_B = 16384
_NW = 16
_B_PER_W = _B // _NW
_G_CHUNKS = _B_PER_W // _L
_BUILD_UNROLL = 4
_GATHER_UNROLL = 4

_mesh = plsc.VectorSubcoreMesh(core_axis_name="c", subcore_axis_name="s", num_cores=1)


@functools.partial(
    pl.kernel,
    mesh=_mesh,
    out_type=jax.ShapeDtypeStruct((_B,), jnp.float32),
    compiler_params=pltpu.CompilerParams(needs_layout_passes=False),
    scratch_types=[
        pltpu.VMEM((_T_PAD,), jnp.float32),
        pltpu.VMEM((2 * _L,), jnp.float32),
        pltpu.VMEM((_B_PER_W,), jnp.int32),
        pltpu.VMEM((_B_PER_W,), jnp.float32),
        pltpu.SemaphoreType.DMA,
        pltpu.SemaphoreType.DMA,
        pltpu.SemaphoreType.DMA,
    ],
)
def _snr_lookup(ts_hbm, w_hbm, wini_hbm, md_hbm, bini_hbm, base_hbm, out_hbm,
                tab_v, scal_v, idx_v, out_v, sem0, sem1, sem2):
    wid = lax.axis_index("s")
    off = wid * _B_PER_W
    cw = pltpu.async_copy(w_hbm, tab_v.at[pl.ds(0, _T)], sem0)
    c0 = pltpu.async_copy(wini_hbm, scal_v.at[pl.ds(0, 1)], sem1)
    c1 = pltpu.async_copy(md_hbm, scal_v.at[pl.ds(8, 1)], sem1)
    c2 = pltpu.async_copy(bini_hbm, scal_v.at[pl.ds(16, 1)], sem1)
    c3 = pltpu.async_copy(base_hbm, scal_v.at[pl.ds(24, 1)], sem1)
    ci = pltpu.async_copy(ts_hbm.at[pl.ds(off, _B_PER_W)], idx_v, sem2)

    c0.wait()
    c1.wait()
    c2.wait()
    c3.wait()
    sv0 = scal_v[pl.ds(0, _L)]
    sv1 = scal_v[pl.ds(_L, _L)]
    wini = lax.broadcast(sv0[0], (_L,))
    md = lax.broadcast(sv0[8], (_L,))
    carry0 = lax.broadcast(-(sv1[0] + sv1[8]), (_L,))
    cw.wait()

    @plsc.parallel_loop(0, _N_CHUNKS, 1, unroll=_BUILD_UNROLL, carry=carry0)
    def _build(i, carry):
        x = tab_v[pl.ds(i * _L, _L)] + wini
        s = 1.0 / (1.0 + jnp.exp(-x)) + md
        tab_v[pl.ds(i * _L, _L)] = plsc.cumsum(s) + carry
        return carry + lax.broadcast(jnp.sum(s), (_L,))

    ci.wait()

    @plsc.parallel_loop(0, _G_CHUNKS, 1, unroll=_GATHER_UNROLL)
    def _gather(j):
        ids = idx_v[pl.ds(j * _L, _L)]
        out_v[pl.ds(j * _L, _L)] = plsc.load_gather(tab_v, [ids])

    pltpu.sync_copy(out_v, out_hbm.at[pl.ds(off, _B_PER_W)])


def kernel(timesteps, W, base, w_ini, base_ini, min_diff):
    as1 = lambda x: jnp.asarray(x, jnp.float32).reshape(1)
    return _snr_lookup(timesteps, W, as1(w_ini), as1(min_diff),
                       as1(base_ini), as1(base[0]))
